# baseline (speedup 1.0000x reference)
import jax
import jax.numpy as jnp
from jax.experimental import pallas as pl
from jax.experimental.pallas import tpu as pltpu

NUM_EXPERTS = 64
TOP_K = 2
ALPHA = 0.01
DIM = 2048
T = 16384

BLK = 2048
NBLK = T // BLK


def _gate_kernel(x_ref, wt_ref, idx_ref, w_ref, pi_ref, cnt_ref):
    logits = jnp.dot(x_ref[...], wt_ref[...],
                     preferred_element_type=jnp.float32)

    col = jax.lax.broadcasted_iota(jnp.int32, logits.shape, 1)
    colrev = ((NUM_EXPERTS - 1) - col).astype(jnp.float32)

    m1 = jnp.max(logits, axis=-1, keepdims=True)
    c1 = logits == m1
    a1 = jnp.max(jnp.where(c1, colrev, -1.0), axis=-1, keepdims=True)
    idx1 = (jnp.float32(NUM_EXPERTS - 1) - a1).astype(jnp.int32)
    masked = jnp.where(c1, -jnp.inf, logits)
    m2 = jnp.max(masked, axis=-1, keepdims=True)
    c2 = masked == m2
    a2 = jnp.max(jnp.where(c2, colrev, -1.0), axis=-1, keepdims=True)
    idx2 = (jnp.float32(NUM_EXPERTS - 1) - a2).astype(jnp.int32)

    r = jnp.exp(m2 - m1)
    w1 = 1.0 / (1.0 + r)
    w2 = 1.0 - w1

    idx_ref[...] = jnp.concatenate([idx1, idx2], axis=1)
    w_ref[...] = jnp.concatenate([w1, w2], axis=1)

    e = jnp.exp(logits - m1)
    s = jnp.sum(e, axis=-1, keepdims=True)
    scores = e * (1.0 / s)

    pi_part = jnp.sum(scores, axis=0, keepdims=True)
    cnt_part = jnp.sum(c1.astype(jnp.float32) + c2.astype(jnp.float32),
                       axis=0, keepdims=True)
    pi_ref[...] = jnp.broadcast_to(pi_part, (8, NUM_EXPERTS))
    cnt_ref[...] = jnp.broadcast_to(cnt_part, (8, NUM_EXPERTS))


def _aux_kernel(pi_ref, cnt_ref, aux_ref):
    pi = jnp.sum(pi_ref[...], axis=0) * jnp.float32(1.0 / (8 * T))
    fi = jnp.sum(cnt_ref[...], axis=0) * jnp.float32(
        NUM_EXPERTS / (8 * T * TOP_K))
    aux_ref[...] = (jnp.sum(pi * fi) * jnp.float32(ALPHA)).reshape(1, 1)


def kernel(x, weight):
    wt = weight.astype(jnp.float32).T
    xf = x.astype(jnp.float32)
    idx, w, pi_p, cnt_p = pl.pallas_call(
        _gate_kernel,
        grid=(NBLK,),
        in_specs=[
            pl.BlockSpec((BLK, DIM), lambda i: (i, 0)),
            pl.BlockSpec((DIM, NUM_EXPERTS), lambda i: (0, 0)),
        ],
        out_specs=[
            pl.BlockSpec((BLK, TOP_K), lambda i: (i, 0)),
            pl.BlockSpec((BLK, TOP_K), lambda i: (i, 0)),
            pl.BlockSpec((8, NUM_EXPERTS), lambda i: (i, 0)),
            pl.BlockSpec((8, NUM_EXPERTS), lambda i: (i, 0)),
        ],
        out_shape=[
            jax.ShapeDtypeStruct((T, TOP_K), jnp.int32),
            jax.ShapeDtypeStruct((T, TOP_K), jnp.float32),
            jax.ShapeDtypeStruct((NBLK * 8, NUM_EXPERTS), jnp.float32),
            jax.ShapeDtypeStruct((NBLK * 8, NUM_EXPERTS), jnp.float32),
        ],
        compiler_params=pltpu.CompilerParams(
            dimension_semantics=("parallel",),
        ),
    )(xf, wt)
    aux = pl.pallas_call(
        _aux_kernel,
        out_shape=jax.ShapeDtypeStruct((1, 1), jnp.float32),
    )(pi_p, cnt_p)
    return (idx, w, aux.reshape(()))


# speedup vs baseline: 1.0200x; 1.0200x over previous
import jax
import jax.numpy as jnp
from jax.experimental import pallas as pl
from jax.experimental.pallas import tpu as pltpu

NUM_EXPERTS = 64
TOP_K = 2
ALPHA = 0.01
DIM = 2048
T = 16384

BLK = 2048


def _gate_kernel(x_ref, wt_ref, idx_ref, w_ref, aux_ref, acc_ref):
    i = pl.program_id(0)
    n = pl.num_programs(0)

    @pl.when(i == 0)
    def _init():
        acc_ref[...] = jnp.zeros_like(acc_ref)

    logits = jnp.dot(x_ref[...], wt_ref[...],
                     preferred_element_type=jnp.float32)

    col = jax.lax.broadcasted_iota(jnp.int32, logits.shape, 1)
    colrev = ((NUM_EXPERTS - 1) - col).astype(jnp.float32)

    m1 = jnp.max(logits, axis=-1, keepdims=True)
    c1 = logits == m1
    a1 = jnp.max(jnp.where(c1, colrev, -1.0), axis=-1, keepdims=True)
    idx1 = (jnp.float32(NUM_EXPERTS - 1) - a1).astype(jnp.int32)
    masked = jnp.where(c1, -jnp.inf, logits)
    m2 = jnp.max(masked, axis=-1, keepdims=True)
    c2 = masked == m2
    a2 = jnp.max(jnp.where(c2, colrev, -1.0), axis=-1, keepdims=True)
    idx2 = (jnp.float32(NUM_EXPERTS - 1) - a2).astype(jnp.int32)

    r = jnp.exp(m2 - m1)
    w1 = 1.0 / (1.0 + r)
    w2 = 1.0 - w1

    idx_ref[...] = jnp.concatenate([idx1, idx2], axis=1)
    w_ref[...] = jnp.concatenate([w1, w2], axis=1)

    e = jnp.exp(logits - m1)
    s = jnp.sum(e, axis=-1, keepdims=True)
    scores = e * (1.0 / s)

    pi_part = jnp.sum(scores, axis=0, keepdims=True)
    cnt_part = jnp.sum(c1.astype(jnp.float32) + c2.astype(jnp.float32),
                       axis=0, keepdims=True)
    acc_ref[0:1, :] += pi_part
    acc_ref[1:2, :] += cnt_part

    @pl.when(i == n - 1)
    def _fin():
        pi = acc_ref[0:1, :] / jnp.float32(T)
        fi = acc_ref[1:2, :] * jnp.float32(NUM_EXPERTS / (T * TOP_K))
        aux_ref[...] = (jnp.sum(pi * fi) * jnp.float32(ALPHA)).reshape(1, 1)


def kernel(x, weight):
    wt = weight.astype(jnp.float32).T
    xf = x.astype(jnp.float32)
    grid = (T // BLK,)
    idx, w, aux = pl.pallas_call(
        _gate_kernel,
        grid=grid,
        in_specs=[
            pl.BlockSpec((BLK, DIM), lambda i: (i, 0)),
            pl.BlockSpec((DIM, NUM_EXPERTS), lambda i: (0, 0)),
        ],
        out_specs=[
            pl.BlockSpec((BLK, TOP_K), lambda i: (i, 0)),
            pl.BlockSpec((BLK, TOP_K), lambda i: (i, 0)),
            pl.BlockSpec((1, 1), lambda i: (0, 0)),
        ],
        out_shape=[
            jax.ShapeDtypeStruct((T, TOP_K), jnp.int32),
            jax.ShapeDtypeStruct((T, TOP_K), jnp.float32),
            jax.ShapeDtypeStruct((1, 1), jnp.float32),
        ],
        scratch_shapes=[pltpu.VMEM((2, NUM_EXPERTS), jnp.float32)],
        compiler_params=pltpu.CompilerParams(
            dimension_semantics=("arbitrary",),
        ),
    )(xf, wt)
    return (idx, w, aux.reshape(()))
